# Initial kernel scaffold; baseline (speedup 1.0000x reference)
import jax
import jax.numpy as jnp
from jax.experimental import pallas as pl


def kernel(embedding_m, memberships, s_idx, o_idx, layer_id, We, Wm, b_c, Ws, bs, Wo, bo):
    raise NotImplementedError("write your pallas kernel here")


# speedup vs baseline: 6.6190x; 6.6190x over previous
import functools

import jax
import jax.numpy as jnp
from jax import lax
from jax.experimental import pallas as pl
from jax.experimental.pallas import tpu as pltpu
from jax.experimental.pallas import tpu_sc as plsc

N = 10000
EMB = 128
E = 320000
NL = 4
NW = 32
EPW = E // NW
CH = 80
NCH = EPW // CH
BN = 1000


def _tc_project_body(e_ref, m_ref, we_ref, wm_ref, bc_ref, ws_ref, wt_ref,
                     bh_ref, upd_ref, ss_ref, so_ref, ts_ref, to_ref):
    u = jnp.tanh(
        jnp.dot(e_ref[...], we_ref[...], preferred_element_type=jnp.float32)
        + jnp.dot(m_ref[...], wm_ref[...], preferred_element_type=jnp.float32)
        + bc_ref[...]
    )
    upd_ref[...] = u
    sb = jnp.dot(u, ws_ref[...], preferred_element_type=jnp.float32) + bh_ref[...]
    tb = jnp.dot(u, wt_ref[...], preferred_element_type=jnp.float32) + bh_ref[...]
    for l in range(NL):
        ss_ref[:, l, :] = sb[:, 256 * l:256 * l + 128]
        so_ref[:, l, :] = sb[:, 256 * l + 128:256 * (l + 1)]
        ts_ref[:, l, :] = tb[:, 256 * l:256 * l + 128]
        to_ref[:, l, :] = tb[:, 256 * l + 128:256 * (l + 1)]


def _tc_project(e, m, we, wm, bc2, ws, wt, bh):
    tbl = jax.ShapeDtypeStruct((N, NL, EMB), jnp.float32)
    tbl_spec = pl.BlockSpec((BN, NL, EMB), lambda i: (i, 0, 0))
    return pl.pallas_call(
        _tc_project_body,
        grid=(N // BN,),
        in_specs=[
            pl.BlockSpec((BN, EMB), lambda i: (i, 0)),
            pl.BlockSpec((BN, 8), lambda i: (i, 0)),
            pl.BlockSpec((EMB, EMB), lambda i: (0, 0)),
            pl.BlockSpec((8, EMB), lambda i: (0, 0)),
            pl.BlockSpec((1, EMB), lambda i: (0, 0)),
            pl.BlockSpec((EMB, NL * 256), lambda i: (0, 0)),
            pl.BlockSpec((EMB, NL * 256), lambda i: (0, 0)),
            pl.BlockSpec((1, NL * 256), lambda i: (0, 0)),
        ],
        out_specs=[pl.BlockSpec((BN, EMB), lambda i: (i, 0)),
                   tbl_spec, tbl_spec, tbl_spec, tbl_spec],
        out_shape=[jax.ShapeDtypeStruct((N, EMB), jnp.float32),
                   tbl, tbl, tbl, tbl],
    )(e, m, we, wm, bc2, ws, wt, bh)


def _tc_combine_body(upd_ref, acc_ref, out_ref):
    a = upd_ref[...] + acc_ref[0] + acc_ref[1]
    nrm = jnp.sqrt(jnp.sum(a * a, axis=1, keepdims=True))
    out_ref[...] = a / jnp.maximum(nrm, 1e-12)


def _tc_combine(upd, acc2):
    return pl.pallas_call(
        _tc_combine_body,
        grid=(N // BN,),
        in_specs=[
            pl.BlockSpec((BN, EMB), lambda i: (i, 0)),
            pl.BlockSpec((2, BN, EMB), lambda i: (0, i, 0)),
        ],
        out_specs=pl.BlockSpec((BN, EMB), lambda i: (i, 0)),
        out_shape=jax.ShapeDtypeStruct((N, EMB), jnp.float32),
    )(upd, acc2)


def _tanh16(x):
    return 1.0 - 2.0 / (jnp.exp(2.0 * x) + 1.0)


def _sc_edge_body(ss_hbm, so_hbm, ts_hbm, to_hbm, sidx_hbm, oidx_hbm,
                  lay_hbm, zacc_hbm, out_hbm,
                  sbuf, obuf, gsbuf, gobuf, buf1, buf2, acc_sh, sem1, sem2):
    cid = lax.axis_index("c")
    sid = lax.axis_index("s")
    wid = sid * 2 + cid

    rows0 = sid * 624
    pltpu.sync_copy(zacc_hbm.at[pl.ds(rows0, 640)],
                    acc_sh.at[pl.ds(rows0, 640)])
    plsc.subcore_barrier()

    def chunk(c, _):
        base = wid * EPW + c * CH
        pltpu.sync_copy(sidx_hbm.at[pl.ds(base, CH)], sbuf)
        pltpu.sync_copy(oidx_hbm.at[pl.ds(base, CH)], obuf)
        pltpu.sync_copy(lay_hbm.at[pl.ds(base, CH)], gobuf)
        for j in range(CH // 16):
            sl = pl.ds(j * 16, 16)
            l = gobuf[sl]
            gsbuf[sl] = sbuf[sl] * 4 + l
            gobuf[sl] = obuf[sl] * 4 + l

        def halfpass(a_hbm, b_hbm, idx):
            cp1 = pltpu.async_copy(a_hbm.at[gsbuf], buf1, sem1)
            cp2 = pltpu.async_copy(b_hbm.at[gobuf], buf2, sem2)
            cp1.wait()
            cp2.wait()

            def row(i, _):
                for j in range(EMB // 16):
                    sl = pl.ds(j * 16, 16)
                    buf1[i, sl] = _tanh16(buf1[i, sl] + buf2[i, sl])
                return 0

            lax.fori_loop(0, CH, row, 0)
            pltpu.sync_copy(buf1, acc_sh.at[idx], add=True)

        halfpass(ss_hbm, ts_hbm, sbuf)
        halfpass(so_hbm, to_hbm, obuf)
        return 0

    lax.fori_loop(0, NCH, chunk, 0)
    plsc.subcore_barrier()

    pltpu.sync_copy(acc_sh.at[pl.ds(rows0, 640)],
                    out_hbm.at[cid, pl.ds(rows0, 640)])


_sc_edge = functools.partial(
    pl.kernel,
    out_type=jax.ShapeDtypeStruct((2, N, EMB), jnp.float32),
    mesh=plsc.VectorSubcoreMesh(core_axis_name="c", subcore_axis_name="s"),
    scratch_types=[
        pltpu.VMEM((CH,), jnp.int32),
        pltpu.VMEM((CH,), jnp.int32),
        pltpu.VMEM((CH,), jnp.int32),
        pltpu.VMEM((CH,), jnp.int32),
        pltpu.VMEM((CH, EMB), jnp.float32),
        pltpu.VMEM((CH, EMB), jnp.float32),
        pltpu.VMEM_SHARED((N, EMB), jnp.float32),
        pltpu.SemaphoreType.DMA,
        pltpu.SemaphoreType.DMA,
    ],
)(_sc_edge_body)


def kernel(embedding_m, memberships, s_idx, o_idx, layer_id, We, Wm, b_c, Ws,
           bs, Wo, bo):
    w_s = jnp.concatenate(
        [jnp.concatenate([Ws[l, :EMB, :], Wo[l, :EMB, :]], axis=1)
         for l in range(NL)], axis=1)
    w_t = jnp.concatenate(
        [jnp.concatenate([Ws[l, EMB:, :], Wo[l, EMB:, :]], axis=1)
         for l in range(NL)], axis=1)
    bh = (jnp.concatenate([bs, bo], axis=1) * 0.5).reshape(1, NL * 256)
    bc2 = b_c.reshape(1, EMB)
    zacc = jnp.zeros((N, EMB), jnp.float32)

    e = embedding_m
    for _t in range(2):
        upd, ss, so, ts, to = _tc_project(e, memberships, We, Wm, bc2,
                                          w_s, w_t, bh)
        acc2 = _sc_edge(ss.reshape(N * NL, EMB), so.reshape(N * NL, EMB),
                        ts.reshape(N * NL, EMB), to.reshape(N * NL, EMB),
                        s_idx, o_idx, layer_id, zacc)
        e = _tc_combine(upd, acc2)
    return e


# speedup vs baseline: 11.1583x; 1.6858x over previous
import functools

import jax
import jax.numpy as jnp
from jax import lax
from jax.experimental import pallas as pl
from jax.experimental.pallas import tpu as pltpu
from jax.experimental.pallas import tpu_sc as plsc

N = 10000
EMB = 128
E = 320000
NL = 4
NW = 32
EPW = E // NW
SW = 2000
NSW = EPW // SW
CH = 80
NCH = SW // CH
BN = 1000


def _tc_project_body(e_ref, m_ref, we_ref, wm_ref, bc_ref, ws_ref, wt_ref,
                     bh_ref, upd_ref, ss_ref, so_ref, ts_ref, to_ref):
    u = jnp.tanh(
        jnp.dot(e_ref[...], we_ref[...], preferred_element_type=jnp.float32)
        + jnp.dot(m_ref[...], wm_ref[...], preferred_element_type=jnp.float32)
        + bc_ref[...]
    )
    upd_ref[...] = u
    sb = jnp.dot(u, ws_ref[...], preferred_element_type=jnp.float32) + bh_ref[...]
    tb = jnp.dot(u, wt_ref[...], preferred_element_type=jnp.float32) + bh_ref[...]
    for l in range(NL):
        ss_ref[:, l, :] = sb[:, 256 * l:256 * l + 128]
        so_ref[:, l, :] = sb[:, 256 * l + 128:256 * (l + 1)]
        ts_ref[:, l, :] = tb[:, 256 * l:256 * l + 128]
        to_ref[:, l, :] = tb[:, 256 * l + 128:256 * (l + 1)]


def _tc_project(e, m, we, wm, bc2, ws, wt, bh):
    tbl = jax.ShapeDtypeStruct((N, NL, EMB), jnp.float32)
    tbl_spec = pl.BlockSpec((BN, NL, EMB), lambda i: (i, 0, 0))
    return pl.pallas_call(
        _tc_project_body,
        grid=(N // BN,),
        in_specs=[
            pl.BlockSpec((BN, EMB), lambda i: (i, 0)),
            pl.BlockSpec((BN, 8), lambda i: (i, 0)),
            pl.BlockSpec((EMB, EMB), lambda i: (0, 0)),
            pl.BlockSpec((8, EMB), lambda i: (0, 0)),
            pl.BlockSpec((1, EMB), lambda i: (0, 0)),
            pl.BlockSpec((EMB, NL * 256), lambda i: (0, 0)),
            pl.BlockSpec((EMB, NL * 256), lambda i: (0, 0)),
            pl.BlockSpec((1, NL * 256), lambda i: (0, 0)),
        ],
        out_specs=[pl.BlockSpec((BN, EMB), lambda i: (i, 0)),
                   tbl_spec, tbl_spec, tbl_spec, tbl_spec],
        out_shape=[jax.ShapeDtypeStruct((N, EMB), jnp.float32),
                   tbl, tbl, tbl, tbl],
    )(e, m, we, wm, bc2, ws, wt, bh)


def _tc_combine_body(upd_ref, acc_ref, out_ref):
    a = upd_ref[...] + acc_ref[0] + acc_ref[1]
    nrm = jnp.sqrt(jnp.sum(a * a, axis=1, keepdims=True))
    out_ref[...] = a / jnp.maximum(nrm, 1e-12)


def _tc_combine(upd, acc2):
    return pl.pallas_call(
        _tc_combine_body,
        grid=(N // BN,),
        in_specs=[
            pl.BlockSpec((BN, EMB), lambda i: (i, 0)),
            pl.BlockSpec((2, BN, EMB), lambda i: (0, i, 0)),
        ],
        out_specs=pl.BlockSpec((BN, EMB), lambda i: (i, 0)),
        out_shape=jax.ShapeDtypeStruct((N, EMB), jnp.float32),
    )(upd, acc2)


def _tanh16(x):
    return 1.0 - 2.0 / (jnp.exp(2.0 * x) + 1.0)


def _sc_edge_body(ss_hbm, so_hbm, ts_hbm, to_hbm, sidx_hbm, oidx_hbm,
                  lay_hbm, zacc_hbm, out_hbm,
                  gsb, gob, a1, a2, b1, b2, sca, scb, acc_sh, sem_a, sem_b):
    cid = lax.axis_index("c")
    sid = lax.axis_index("s")
    wid = sid * 2 + cid

    rows0 = sid * 624
    pltpu.sync_copy(zacc_hbm.at[pl.ds(rows0, 640)],
                    acc_sh.at[pl.ds(rows0, 640)])
    plsc.subcore_barrier()

    def fire_a(c):
        pltpu.async_copy(ss_hbm.at[gsb.at[pl.ds(c * CH, CH)]], a1, sem_a)
        pltpu.async_copy(ts_hbm.at[gob.at[pl.ds(c * CH, CH)]], a2, sem_a)

    def fire_b(c):
        pltpu.async_copy(so_hbm.at[gsb.at[pl.ds(c * CH, CH)]], b1, sem_b)
        pltpu.async_copy(to_hbm.at[gob.at[pl.ds(c * CH, CH)]], b2, sem_b)

    def wait_pair(d1, d2, sem):
        pltpu.make_async_copy(ss_hbm.at[pl.ds(0, CH)], d1, sem).wait()
        pltpu.make_async_copy(ss_hbm.at[pl.ds(0, CH)], d2, sem).wait()

    def sweep(w, _):
        base = wid * EPW + w * SW
        pltpu.sync_copy(sidx_hbm.at[pl.ds(base, SW)], gsb)
        pltpu.sync_copy(lay_hbm.at[pl.ds(base, SW)], gob)

        def build1(k, _):
            sl = pl.ds(k * 16, 16)
            gsb[sl] = gsb[sl] * 4 + gob[sl]
            return 0

        lax.fori_loop(0, SW // 16, build1, 0)
        pltpu.sync_copy(oidx_hbm.at[pl.ds(base, SW)], gob)

        def build2(k, _):
            sl = pl.ds(k * 16, 16)
            gob[sl] = gob[sl] * 4 + (gsb[sl] & 3)
            return 0

        lax.fori_loop(0, SW // 16, build2, 0)

        fire_a(0)
        fire_b(0)

        def chunk(c, _):
            wait_pair(a1, a2, sem_a)
            for j in range(CH // 16):
                sca[pl.ds(j * 16, 16)] = (
                    gsb[pl.ds(c * CH + j * 16, 16)] >> 2)

            def row_a(i, _):
                for j in range(EMB // 16):
                    sl = pl.ds(j * 16, 16)
                    a1[i, sl] = _tanh16(a1[i, sl] + a2[i, sl])
                return 0

            lax.fori_loop(0, CH, row_a, 0)
            pltpu.sync_copy(a1, acc_sh.at[sca], add=True)

            @pl.when(c < NCH - 1)
            def _():
                fire_a(c + 1)

            wait_pair(b1, b2, sem_b)
            for j in range(CH // 16):
                scb[pl.ds(j * 16, 16)] = (
                    gob[pl.ds(c * CH + j * 16, 16)] >> 2)

            def row_b(i, _):
                for j in range(EMB // 16):
                    sl = pl.ds(j * 16, 16)
                    b1[i, sl] = _tanh16(b1[i, sl] + b2[i, sl])
                return 0

            lax.fori_loop(0, CH, row_b, 0)
            pltpu.sync_copy(b1, acc_sh.at[scb], add=True)

            @pl.when(c < NCH - 1)
            def _():
                fire_b(c + 1)

            return 0

        lax.fori_loop(0, NCH, chunk, 0)
        return 0

    lax.fori_loop(0, NSW, sweep, 0)
    plsc.subcore_barrier()

    pltpu.sync_copy(acc_sh.at[pl.ds(rows0, 640)],
                    out_hbm.at[cid, pl.ds(rows0, 640)])


_sc_edge = functools.partial(
    pl.kernel,
    out_type=jax.ShapeDtypeStruct((2, N, EMB), jnp.float32),
    mesh=plsc.VectorSubcoreMesh(core_axis_name="c", subcore_axis_name="s"),
    scratch_types=[
        pltpu.VMEM((SW,), jnp.int32),
        pltpu.VMEM((SW,), jnp.int32),
        pltpu.VMEM((CH, EMB), jnp.float32),
        pltpu.VMEM((CH, EMB), jnp.float32),
        pltpu.VMEM((CH, EMB), jnp.float32),
        pltpu.VMEM((CH, EMB), jnp.float32),
        pltpu.VMEM((CH,), jnp.int32),
        pltpu.VMEM((CH,), jnp.int32),
        pltpu.VMEM_SHARED((N, EMB), jnp.float32),
        pltpu.SemaphoreType.DMA,
        pltpu.SemaphoreType.DMA,
    ],
)(_sc_edge_body)


def kernel(embedding_m, memberships, s_idx, o_idx, layer_id, We, Wm, b_c, Ws,
           bs, Wo, bo):
    w_s = jnp.concatenate(
        [jnp.concatenate([Ws[l, :EMB, :], Wo[l, :EMB, :]], axis=1)
         for l in range(NL)], axis=1)
    w_t = jnp.concatenate(
        [jnp.concatenate([Ws[l, EMB:, :], Wo[l, EMB:, :]], axis=1)
         for l in range(NL)], axis=1)
    bh = (jnp.concatenate([bs, bo], axis=1) * 0.5).reshape(1, NL * 256)
    bc2 = b_c.reshape(1, EMB)
    zacc = jnp.zeros((N, EMB), jnp.float32)

    e = embedding_m
    for _t in range(2):
        upd, ss, so, ts, to = _tc_project(e, memberships, We, Wm, bc2,
                                          w_s, w_t, bh)
        acc2 = _sc_edge(ss.reshape(N * NL, EMB), so.reshape(N * NL, EMB),
                        ts.reshape(N * NL, EMB), to.reshape(N * NL, EMB),
                        s_idx, o_idx, layer_id, zacc)
        e = _tc_combine(upd, acc2)
    return e
